# baseline (speedup 1.0000x reference)
import jax
import jax.numpy as jnp
from jax import lax
from jax.experimental import pallas as pl
from jax.experimental.pallas import tpu as pltpu
from jax.experimental.pallas import tpu_sc as plsc

NN = 10000
EE = 160000
DD = 128
RR = 16
INV_T_C = 0.1

NTILES = 32
CHUNK = 80
DCH = 128
EPAD = 163840
EPT = EPAD // NTILES
NCH = EPT // CHUNK
NCHD = EPT // DCH
NPAD = 10240
RPT = NPAD // 16

BN = 400


def _fill_zero(buf, nrows):
    z16 = jnp.zeros((16,), jnp.float32)

    def _fill(i, _):
        for j in range(DD // 16):
            buf[i, pl.ds(j * 16, 16)] = z16
        return 0

    lax.fori_loop(0, nrows, _fill, 0)


def _zero_spmem(sh, zbuf, r0, step):
    def _z(k, _):
        pltpu.sync_copy(zbuf, sh.at[pl.ds(r0 + k * step, step)])
        return 0

    lax.fori_loop(0, RPT // step, _z, 0)


def _copy_out(sh, zbuf, out, r0, ob, step):
    def _c(k, _):
        pltpu.sync_copy(sh.at[pl.ds(r0 + k * step, step)], zbuf)
        pltpu.sync_copy(zbuf, out.at[pl.ds(ob + k * step, step)])
        return 0

    lax.fori_loop(0, RPT // step, _c, 0)


def _sc_agg_body(z_hbm, gidx_hbm, dst_hbm, agg_out,
                 agg_sh, gv0, gv1, dv0, dv1, rows0, rows1, zbuf,
                 ms0, ms1, gs0, gs1, ss0, ss1):
    cid = lax.axis_index("c")
    sid = lax.axis_index("s")
    wid = sid * 2 + cid

    _fill_zero(zbuf, 8)
    r0 = sid * RPT
    _zero_spmem(agg_sh, zbuf, r0, 8)
    plsc.subcore_barrier()

    base = wid * EPT
    gv = (gv0, gv1)
    dv = (dv0, dv1)
    rows = (rows0, rows1)
    msem = (ms0, ms1)
    gsem = (gs0, gs1)
    ssem = (ss0, ss1)

    def _prep(c, b):
        off = base + c * CHUNK
        pltpu.async_copy(gidx_hbm.at[pl.ds(off, CHUNK)], gv[b], msem[b])
        pltpu.async_copy(dst_hbm.at[pl.ds(off, CHUNK)], dv[b], msem[b])
        pltpu.make_async_copy(gidx_hbm.at[pl.ds(off, CHUNK)], gv[b], msem[b]).wait()
        pltpu.make_async_copy(dst_hbm.at[pl.ds(off, CHUNK)], dv[b], msem[b]).wait()
        pltpu.async_copy(z_hbm.at[gv[b]], rows[b], gsem[b])

    for b in range(2):
        _prep(b, b)

    def _pipe(g, _):
        for b in range(2):
            pltpu.make_async_copy(z_hbm.at[gv[b]], rows[b], gsem[b]).wait()
            pltpu.async_copy(rows[b], agg_sh.at[dv[b]], ssem[b], add=True)
        for b in range(2):
            pltpu.make_async_copy(rows[b], agg_sh.at[dv[b]], ssem[b]).wait()
            _prep(2 * g + 2 + b, b)
        return 0

    lax.fori_loop(0, NCH // 2 - 1, _pipe, 0)
    for b in range(2):
        pltpu.make_async_copy(z_hbm.at[gv[b]], rows[b], gsem[b]).wait()
        pltpu.async_copy(rows[b], agg_sh.at[dv[b]], ssem[b], add=True)
    for b in range(2):
        pltpu.make_async_copy(rows[b], agg_sh.at[dv[b]], ssem[b]).wait()

    plsc.subcore_barrier()
    _copy_out(agg_sh, zbuf, agg_out, r0, cid * NPAD + r0, 8)


_sc_agg = pl.kernel(
    _sc_agg_body,
    out_type=jax.ShapeDtypeStruct((2 * NPAD, DD), jnp.float32),
    mesh=plsc.VectorSubcoreMesh(core_axis_name="c", subcore_axis_name="s"),
    scratch_types=[
        pltpu.VMEM_SHARED((NPAD, DD), jnp.float32),
        pltpu.VMEM((CHUNK,), jnp.int32),
        pltpu.VMEM((CHUNK,), jnp.int32),
        pltpu.VMEM((CHUNK,), jnp.int32),
        pltpu.VMEM((CHUNK,), jnp.int32),
        pltpu.VMEM((CHUNK, DD), jnp.float32),
        pltpu.VMEM((CHUNK, DD), jnp.float32),
        pltpu.VMEM((8, DD), jnp.float32),
        pltpu.SemaphoreType.DMA,
        pltpu.SemaphoreType.DMA,
        pltpu.SemaphoreType.DMA,
        pltpu.SemaphoreType.DMA,
        pltpu.SemaphoreType.DMA,
        pltpu.SemaphoreType.DMA,
    ],
)


def _sc_deg_body(dst_hbm, deg_out, deg_sh, dstv, ones, zbuf):
    cid = lax.axis_index("c")
    sid = lax.axis_index("s")
    wid = sid * 2 + cid

    _fill_zero(zbuf, 32)
    o16 = jnp.full((16,), 1.0, jnp.float32)

    def _fill_o(i, _):
        for j in range(DD // 16):
            ones[i, pl.ds(j * 16, 16)] = o16
        return 0

    lax.fori_loop(0, DCH, _fill_o, 0)

    r0 = sid * RPT
    _zero_spmem(deg_sh, zbuf, r0, 32)
    plsc.subcore_barrier()

    base = wid * EPT

    def _edge_chunk(c, _):
        off = base + c * DCH
        pltpu.sync_copy(dst_hbm.at[pl.ds(off, DCH)], dstv)
        pltpu.sync_copy(ones, deg_sh.at[dstv], add=True)
        return 0

    lax.fori_loop(0, NCHD, _edge_chunk, 0)
    plsc.subcore_barrier()
    _copy_out(deg_sh, zbuf, deg_out, r0, cid * NPAD + r0, 32)


_sc_deg = pl.kernel(
    _sc_deg_body,
    out_type=jax.ShapeDtypeStruct((2 * NPAD, DD), jnp.float32),
    mesh=plsc.VectorSubcoreMesh(core_axis_name="c", subcore_axis_name="s"),
    scratch_types=[
        pltpu.VMEM_SHARED((NPAD, DD), jnp.float32),
        pltpu.VMEM((DCH,), jnp.int32),
        pltpu.VMEM((DCH, DD), jnp.float32),
        pltpu.VMEM((32, DD), jnp.float32),
    ],
)


def _gidx_body(src_ref, et_ref, out_ref):
    out_ref[...] = et_ref[...] * NN + src_ref[...]


def _gidx_build(src_p, et_p):
    g = pl.pallas_call(
        _gidx_body,
        out_shape=jax.ShapeDtypeStruct((EPAD // 128, 128), jnp.int32),
    )(src_p.reshape(EPAD // 128, 128), et_p.reshape(EPAD // 128, 128))
    return g.reshape(EPAD)


def _zbuild_body(ft_ref, basis_ref, c0_ref, c1_ref, z_ref):
    ft = ft_ref[...]
    xb0 = jnp.dot(ft, basis_ref[0], preferred_element_type=jnp.float32)
    xb1 = jnp.dot(ft, basis_ref[1], preferred_element_type=jnp.float32)
    for r in range(RR):
        z_ref[r] = c0_ref[r] * xb0 + c1_ref[r] * xb1


def _zbuild(ft, basis, coeff):
    c0 = coeff[:, 0]
    c1 = coeff[:, 1]
    z = pl.pallas_call(
        _zbuild_body,
        grid=(NN // BN,),
        in_specs=[
            pl.BlockSpec((BN, DD), lambda i: (i, 0)),
            pl.BlockSpec((2, DD, DD), lambda i: (0, 0, 0)),
            pl.BlockSpec(memory_space=pltpu.SMEM),
            pl.BlockSpec(memory_space=pltpu.SMEM),
        ],
        out_specs=pl.BlockSpec((RR, BN, DD), lambda i: (0, i, 0)),
        out_shape=jax.ShapeDtypeStruct((RR, NN, DD), jnp.float32),
    )(ft, basis, c0, c1)
    return z.reshape(RR * NN, DD)


def _gru_math(pre, adj, wx_ref, wh_ref, bx_ref, bh_ref):
    gi = jnp.dot(pre, wx_ref[...], preferred_element_type=jnp.float32) + bx_ref[...][None, :]
    gh = jnp.dot(adj, wh_ref[...], preferred_element_type=jnp.float32) + bh_ref[...][None, :]
    r = jax.nn.sigmoid(gi[:, :DD] + gh[:, :DD])
    zg = jax.nn.sigmoid(gi[:, DD:2 * DD] + gh[:, DD:2 * DD])
    n = jnp.tanh(gi[:, 2 * DD:] + r * gh[:, 2 * DD:])
    return (1.0 - zg) * n + zg * adj


def _agg_pre(aggp_ref, degp_ref, ft, wself_ref, bias_ref):
    agg = (aggp_ref[0] + aggp_ref[1]) / jnp.maximum(
        degp_ref[0, :, 0:1] + degp_ref[1, :, 0:1], 1.0)
    sf = jnp.dot(ft, wself_ref[...], preferred_element_type=jnp.float32)
    return jax.nn.relu(agg + sf + bias_ref[...][None, :])


def _dense0_body(aggp_ref, degp_ref, ft_ref, prev_ref, td_ref,
                 wself_ref, bias_ref, wx_ref, wh_ref, bx_ref, bh_ref,
                 h_ref, ftn_ref):
    ft = ft_ref[...]
    pre = _agg_pre(aggp_ref, degp_ref, ft, wself_ref, bias_ref)
    adj = prev_ref[...] * jnp.exp(-td_ref[...] * INV_T_C)
    h_ref[...] = _gru_math(pre, adj, wx_ref, wh_ref, bx_ref, bh_ref)
    ftn_ref[...] = pre + ft


def _dense1_body(aggp_ref, degp_ref, ft_ref, prev_ref, td_ref, h0_ref,
                 alpha_ref, wself_ref, bias_ref, wx_ref, wh_ref, bx_ref,
                 bh_ref, h_ref, fin_ref):
    ft = ft_ref[...]
    pre = _agg_pre(aggp_ref, degp_ref, ft, wself_ref, bias_ref)
    adj = prev_ref[...] * jnp.exp(-td_ref[...] * INV_T_C)
    h = _gru_math(pre, adj, wx_ref, wh_ref, bx_ref, bh_ref)
    h_ref[...] = h
    a0 = alpha_ref[0]
    a1 = alpha_ref[1]
    m = jnp.maximum(a0, a1)
    e0 = jnp.exp(a0 - m)
    e1 = jnp.exp(a1 - m)
    w0 = e0 / (e0 + e1)
    fin_ref[...] = w0 * h0_ref[...] + (1.0 - w0) * h


def _full(shape):
    nd = len(shape)
    return pl.BlockSpec(shape, lambda i, _nd=nd: (0,) * _nd)


def _dense0(aggp, degp, ft, prev, td, wself, bias, wx, wh, bx, bh):
    return pl.pallas_call(
        _dense0_body,
        grid=(NN // BN,),
        in_specs=[
            pl.BlockSpec((2, BN, DD), lambda i: (0, i, 0)),
            pl.BlockSpec((2, BN, DD), lambda i: (0, i, 0)),
            pl.BlockSpec((BN, DD), lambda i: (i, 0)),
            pl.BlockSpec((BN, DD), lambda i: (i, 0)),
            pl.BlockSpec((BN, 1), lambda i: (i, 0)),
            _full((DD, DD)),
            _full((DD,)),
            _full((DD, 3 * DD)),
            _full((DD, 3 * DD)),
            _full((3 * DD,)),
            _full((3 * DD,)),
        ],
        out_specs=[
            pl.BlockSpec((BN, DD), lambda i: (i, 0)),
            pl.BlockSpec((BN, DD), lambda i: (i, 0)),
        ],
        out_shape=[
            jax.ShapeDtypeStruct((NN, DD), jnp.float32),
            jax.ShapeDtypeStruct((NN, DD), jnp.float32),
        ],
    )(aggp, degp, ft, prev, td, wself, bias, wx, wh, bx, bh)


def _dense1(aggp, degp, ft, prev, td, h0, alpha, wself, bias, wx, wh, bx, bh):
    return pl.pallas_call(
        _dense1_body,
        grid=(NN // BN,),
        in_specs=[
            pl.BlockSpec((2, BN, DD), lambda i: (0, i, 0)),
            pl.BlockSpec((2, BN, DD), lambda i: (0, i, 0)),
            pl.BlockSpec((BN, DD), lambda i: (i, 0)),
            pl.BlockSpec((BN, DD), lambda i: (i, 0)),
            pl.BlockSpec((BN, 1), lambda i: (i, 0)),
            pl.BlockSpec((BN, DD), lambda i: (i, 0)),
            pl.BlockSpec(memory_space=pltpu.SMEM),
            _full((DD, DD)),
            _full((DD,)),
            _full((DD, 3 * DD)),
            _full((DD, 3 * DD)),
            _full((3 * DD,)),
            _full((3 * DD,)),
        ],
        out_specs=[
            pl.BlockSpec((BN, DD), lambda i: (i, 0)),
            pl.BlockSpec((BN, DD), lambda i: (i, 0)),
        ],
        out_shape=[
            jax.ShapeDtypeStruct((NN, DD), jnp.float32),
            jax.ShapeDtypeStruct((NN, DD), jnp.float32),
        ],
    )(aggp, degp, ft, prev, td, h0, alpha, wself, bias, wx, wh, bx, bh)


def kernel(x, edge_index, edge_type, prev_embed_0, prev_embed_1, time_diff,
           prev_tr_0, prev_tr_1, local_attn_mask, basis_0, coeff_0, wself_0,
           bias_0, gru_wx_0, gru_wh_0, gru_bx_0, gru_bh_0, basis_1, coeff_1,
           wself_1, bias_1, gru_wx_1, gru_wh_1, gru_bx_1, gru_bh_1, lf_alpha):
    src = edge_index[0]
    dst = edge_index[1]
    padlen = EPAD - EE
    src_p = jnp.concatenate([src, jnp.zeros((padlen,), jnp.int32)])
    dst_p = jnp.concatenate([dst, jnp.full((padlen,), NN, jnp.int32)])
    et_p = jnp.concatenate([edge_type, jnp.zeros((padlen,), jnp.int32)])

    degp = _sc_deg(dst_p).reshape(2, NPAD, DD)[:, :NN]
    gidx = _gidx_build(src_p, et_p)

    z0 = _zbuild(x, basis_0, coeff_0)
    aggp0 = _sc_agg(z0, gidx, dst_p).reshape(2, NPAD, DD)[:, :NN]
    h0, ft = _dense0(aggp0, degp, x, prev_embed_0, time_diff,
                     wself_0, bias_0, gru_wx_0, gru_wh_0, gru_bx_0, gru_bh_0)

    z1 = _zbuild(ft, basis_1, coeff_1)
    aggp1 = _sc_agg(z1, gidx, dst_p).reshape(2, NPAD, DD)[:, :NN]
    h1, final = _dense1(aggp1, degp, ft, prev_embed_1, time_diff, h0,
                        lf_alpha, wself_1, bias_1, gru_wx_1, gru_wh_1,
                        gru_bx_1, gru_bh_1)

    return (jnp.stack([h0, h1], axis=0), final)


# speedup vs baseline: 1.0362x; 1.0362x over previous
import jax
import jax.numpy as jnp
from jax import lax
from jax.experimental import pallas as pl
from jax.experimental.pallas import tpu as pltpu
from jax.experimental.pallas import tpu_sc as plsc

NN = 10000
EE = 160000
DD = 128
RR = 16
INV_T_C = 0.1

NTILES = 32
CHUNK = 80
DCH = 128
EPAD = 163840
EPT = EPAD // NTILES
NCH = EPT // CHUNK
NCHD = EPT // DCH
NPAD = 10240
RPT = NPAD // 16

BN = 400


def _fill_zero(buf, nrows):
    z16 = jnp.zeros((16,), jnp.float32)

    def _fill(i, _):
        for j in range(DD // 16):
            buf[i, pl.ds(j * 16, 16)] = z16
        return 0

    lax.fori_loop(0, nrows, _fill, 0)


def _zero_spmem(sh, zbuf, r0, step):
    def _z(k, _):
        pltpu.sync_copy(zbuf, sh.at[pl.ds(r0 + k * step, step)])
        return 0

    lax.fori_loop(0, RPT // step, _z, 0)


def _copy_out(sh, zbuf, out, r0, ob, step):
    def _c(k, _):
        pltpu.sync_copy(sh.at[pl.ds(r0 + k * step, step)], zbuf)
        pltpu.sync_copy(zbuf, out.at[pl.ds(ob + k * step, step)])
        return 0

    lax.fori_loop(0, RPT // step, _c, 0)


def _sc_agg_body(z_hbm, gidx_hbm, dst_hbm, agg_out,
                 agg_sh, gv0, gv1, dv0, dv1, rows0, rows1, zbuf,
                 ms0, ms1, gs0, gs1, ss0, ss1):
    cid = lax.axis_index("c")
    sid = lax.axis_index("s")
    wid = sid * 2 + cid

    _fill_zero(zbuf, 32)
    r0 = sid * RPT
    _zero_spmem(agg_sh, zbuf, r0, 32)
    plsc.subcore_barrier()

    base = wid * EPT
    gv = (gv0, gv1)
    dv = (dv0, dv1)
    rows = (rows0, rows1)
    msem = (ms0, ms1)
    gsem = (gs0, gs1)
    ssem = (ss0, ss1)

    def _prep(c, b):
        off = base + c * CHUNK
        pltpu.async_copy(gidx_hbm.at[pl.ds(off, CHUNK)], gv[b], msem[b])
        pltpu.async_copy(dst_hbm.at[pl.ds(off, CHUNK)], dv[b], msem[b])
        pltpu.make_async_copy(gidx_hbm.at[pl.ds(off, CHUNK)], gv[b], msem[b]).wait()
        pltpu.make_async_copy(dst_hbm.at[pl.ds(off, CHUNK)], dv[b], msem[b]).wait()
        pltpu.async_copy(z_hbm.at[gv[b]], rows[b], gsem[b])

    for b in range(2):
        _prep(b, b)

    def _pipe(g, _):
        for b in range(2):
            pltpu.make_async_copy(z_hbm.at[gv[b]], rows[b], gsem[b]).wait()
            pltpu.async_copy(rows[b], agg_sh.at[dv[b]], ssem[b], add=True)
        for b in range(2):
            pltpu.make_async_copy(rows[b], agg_sh.at[dv[b]], ssem[b]).wait()
            _prep(2 * g + 2 + b, b)
        return 0

    lax.fori_loop(0, NCH // 2 - 1, _pipe, 0)
    for b in range(2):
        pltpu.make_async_copy(z_hbm.at[gv[b]], rows[b], gsem[b]).wait()
        pltpu.async_copy(rows[b], agg_sh.at[dv[b]], ssem[b], add=True)
    for b in range(2):
        pltpu.make_async_copy(rows[b], agg_sh.at[dv[b]], ssem[b]).wait()

    plsc.subcore_barrier()
    _copy_out(agg_sh, zbuf, agg_out, r0, cid * NPAD + r0, 32)


_sc_agg = pl.kernel(
    _sc_agg_body,
    out_type=jax.ShapeDtypeStruct((2 * NPAD, DD), jnp.float32),
    mesh=plsc.VectorSubcoreMesh(core_axis_name="c", subcore_axis_name="s"),
    scratch_types=[
        pltpu.VMEM_SHARED((NPAD, DD), jnp.float32),
        pltpu.VMEM((CHUNK,), jnp.int32),
        pltpu.VMEM((CHUNK,), jnp.int32),
        pltpu.VMEM((CHUNK,), jnp.int32),
        pltpu.VMEM((CHUNK,), jnp.int32),
        pltpu.VMEM((CHUNK, DD), jnp.float32),
        pltpu.VMEM((CHUNK, DD), jnp.float32),
        pltpu.VMEM((32, DD), jnp.float32),
        pltpu.SemaphoreType.DMA,
        pltpu.SemaphoreType.DMA,
        pltpu.SemaphoreType.DMA,
        pltpu.SemaphoreType.DMA,
        pltpu.SemaphoreType.DMA,
        pltpu.SemaphoreType.DMA,
    ],
)


def _sc_deg_body(dst_hbm, deg_out, deg_sh, dstv, ones, zbuf):
    cid = lax.axis_index("c")
    sid = lax.axis_index("s")
    wid = sid * 2 + cid

    _fill_zero(zbuf, 32)
    o16 = jnp.full((16,), 1.0, jnp.float32)

    def _fill_o(i, _):
        for j in range(DD // 16):
            ones[i, pl.ds(j * 16, 16)] = o16
        return 0

    lax.fori_loop(0, DCH, _fill_o, 0)

    r0 = sid * RPT
    _zero_spmem(deg_sh, zbuf, r0, 32)
    plsc.subcore_barrier()

    base = wid * EPT

    def _edge_chunk(c, _):
        off = base + c * DCH
        pltpu.sync_copy(dst_hbm.at[pl.ds(off, DCH)], dstv)
        pltpu.sync_copy(ones, deg_sh.at[dstv], add=True)
        return 0

    lax.fori_loop(0, NCHD, _edge_chunk, 0)
    plsc.subcore_barrier()
    _copy_out(deg_sh, zbuf, deg_out, r0, cid * NPAD + r0, 32)


_sc_deg = pl.kernel(
    _sc_deg_body,
    out_type=jax.ShapeDtypeStruct((2 * NPAD, DD), jnp.float32),
    mesh=plsc.VectorSubcoreMesh(core_axis_name="c", subcore_axis_name="s"),
    scratch_types=[
        pltpu.VMEM_SHARED((NPAD, DD), jnp.float32),
        pltpu.VMEM((DCH,), jnp.int32),
        pltpu.VMEM((DCH, DD), jnp.float32),
        pltpu.VMEM((32, DD), jnp.float32),
    ],
)


def _gidx_body(src_ref, et_ref, out_ref):
    out_ref[...] = et_ref[...] * NN + src_ref[...]


def _gidx_build(src_p, et_p):
    g = pl.pallas_call(
        _gidx_body,
        out_shape=jax.ShapeDtypeStruct((EPAD // 128, 128), jnp.int32),
    )(src_p.reshape(EPAD // 128, 128), et_p.reshape(EPAD // 128, 128))
    return g.reshape(EPAD)


def _zbuild_body(ft_ref, basis_ref, c0_ref, c1_ref, z_ref):
    ft = ft_ref[...]
    xb0 = jnp.dot(ft, basis_ref[0], preferred_element_type=jnp.float32)
    xb1 = jnp.dot(ft, basis_ref[1], preferred_element_type=jnp.float32)
    for r in range(RR):
        z_ref[r] = c0_ref[r] * xb0 + c1_ref[r] * xb1


def _zbuild(ft, basis, coeff):
    c0 = coeff[:, 0]
    c1 = coeff[:, 1]
    z = pl.pallas_call(
        _zbuild_body,
        grid=(NN // BN,),
        in_specs=[
            pl.BlockSpec((BN, DD), lambda i: (i, 0)),
            pl.BlockSpec((2, DD, DD), lambda i: (0, 0, 0)),
            pl.BlockSpec(memory_space=pltpu.SMEM),
            pl.BlockSpec(memory_space=pltpu.SMEM),
        ],
        out_specs=pl.BlockSpec((RR, BN, DD), lambda i: (0, i, 0)),
        out_shape=jax.ShapeDtypeStruct((RR, NN, DD), jnp.float32),
    )(ft, basis, c0, c1)
    return z.reshape(RR * NN, DD)


def _gru_math(pre, adj, wx_ref, wh_ref, bx_ref, bh_ref):
    gi = jnp.dot(pre, wx_ref[...], preferred_element_type=jnp.float32) + bx_ref[...][None, :]
    gh = jnp.dot(adj, wh_ref[...], preferred_element_type=jnp.float32) + bh_ref[...][None, :]
    r = jax.nn.sigmoid(gi[:, :DD] + gh[:, :DD])
    zg = jax.nn.sigmoid(gi[:, DD:2 * DD] + gh[:, DD:2 * DD])
    n = jnp.tanh(gi[:, 2 * DD:] + r * gh[:, 2 * DD:])
    return (1.0 - zg) * n + zg * adj


def _agg_pre(aggp_ref, degp_ref, ft, wself_ref, bias_ref):
    agg = (aggp_ref[0] + aggp_ref[1]) / jnp.maximum(
        degp_ref[0, :, 0:1] + degp_ref[1, :, 0:1], 1.0)
    sf = jnp.dot(ft, wself_ref[...], preferred_element_type=jnp.float32)
    return jax.nn.relu(agg + sf + bias_ref[...][None, :])


def _dense0_body(aggp_ref, degp_ref, ft_ref, prev_ref, td_ref,
                 wself_ref, bias_ref, wx_ref, wh_ref, bx_ref, bh_ref,
                 h_ref, ftn_ref):
    ft = ft_ref[...]
    pre = _agg_pre(aggp_ref, degp_ref, ft, wself_ref, bias_ref)
    adj = prev_ref[...] * jnp.exp(-td_ref[...] * INV_T_C)
    h_ref[...] = _gru_math(pre, adj, wx_ref, wh_ref, bx_ref, bh_ref)
    ftn_ref[...] = pre + ft


def _dense1_body(aggp_ref, degp_ref, ft_ref, prev_ref, td_ref, h0_ref,
                 alpha_ref, wself_ref, bias_ref, wx_ref, wh_ref, bx_ref,
                 bh_ref, h_ref, fin_ref):
    ft = ft_ref[...]
    pre = _agg_pre(aggp_ref, degp_ref, ft, wself_ref, bias_ref)
    adj = prev_ref[...] * jnp.exp(-td_ref[...] * INV_T_C)
    h = _gru_math(pre, adj, wx_ref, wh_ref, bx_ref, bh_ref)
    h_ref[...] = h
    a0 = alpha_ref[0]
    a1 = alpha_ref[1]
    m = jnp.maximum(a0, a1)
    e0 = jnp.exp(a0 - m)
    e1 = jnp.exp(a1 - m)
    w0 = e0 / (e0 + e1)
    fin_ref[...] = w0 * h0_ref[...] + (1.0 - w0) * h


def _full(shape):
    nd = len(shape)
    return pl.BlockSpec(shape, lambda i, _nd=nd: (0,) * _nd)


def _dense0(aggp, degp, ft, prev, td, wself, bias, wx, wh, bx, bh):
    return pl.pallas_call(
        _dense0_body,
        grid=(NN // BN,),
        in_specs=[
            pl.BlockSpec((2, BN, DD), lambda i: (0, i, 0)),
            pl.BlockSpec((2, BN, DD), lambda i: (0, i, 0)),
            pl.BlockSpec((BN, DD), lambda i: (i, 0)),
            pl.BlockSpec((BN, DD), lambda i: (i, 0)),
            pl.BlockSpec((BN, 1), lambda i: (i, 0)),
            _full((DD, DD)),
            _full((DD,)),
            _full((DD, 3 * DD)),
            _full((DD, 3 * DD)),
            _full((3 * DD,)),
            _full((3 * DD,)),
        ],
        out_specs=[
            pl.BlockSpec((BN, DD), lambda i: (i, 0)),
            pl.BlockSpec((BN, DD), lambda i: (i, 0)),
        ],
        out_shape=[
            jax.ShapeDtypeStruct((NN, DD), jnp.float32),
            jax.ShapeDtypeStruct((NN, DD), jnp.float32),
        ],
    )(aggp, degp, ft, prev, td, wself, bias, wx, wh, bx, bh)


def _dense1(aggp, degp, ft, prev, td, h0, alpha, wself, bias, wx, wh, bx, bh):
    return pl.pallas_call(
        _dense1_body,
        grid=(NN // BN,),
        in_specs=[
            pl.BlockSpec((2, BN, DD), lambda i: (0, i, 0)),
            pl.BlockSpec((2, BN, DD), lambda i: (0, i, 0)),
            pl.BlockSpec((BN, DD), lambda i: (i, 0)),
            pl.BlockSpec((BN, DD), lambda i: (i, 0)),
            pl.BlockSpec((BN, 1), lambda i: (i, 0)),
            pl.BlockSpec((BN, DD), lambda i: (i, 0)),
            pl.BlockSpec(memory_space=pltpu.SMEM),
            _full((DD, DD)),
            _full((DD,)),
            _full((DD, 3 * DD)),
            _full((DD, 3 * DD)),
            _full((3 * DD,)),
            _full((3 * DD,)),
        ],
        out_specs=[
            pl.BlockSpec((BN, DD), lambda i: (i, 0)),
            pl.BlockSpec((BN, DD), lambda i: (i, 0)),
        ],
        out_shape=[
            jax.ShapeDtypeStruct((NN, DD), jnp.float32),
            jax.ShapeDtypeStruct((NN, DD), jnp.float32),
        ],
    )(aggp, degp, ft, prev, td, h0, alpha, wself, bias, wx, wh, bx, bh)


def kernel(x, edge_index, edge_type, prev_embed_0, prev_embed_1, time_diff,
           prev_tr_0, prev_tr_1, local_attn_mask, basis_0, coeff_0, wself_0,
           bias_0, gru_wx_0, gru_wh_0, gru_bx_0, gru_bh_0, basis_1, coeff_1,
           wself_1, bias_1, gru_wx_1, gru_wh_1, gru_bx_1, gru_bh_1, lf_alpha):
    src = edge_index[0]
    dst = edge_index[1]
    padlen = EPAD - EE
    src_p = jnp.concatenate([src, jnp.zeros((padlen,), jnp.int32)])
    dst_p = jnp.concatenate([dst, jnp.full((padlen,), NN, jnp.int32)])
    et_p = jnp.concatenate([edge_type, jnp.zeros((padlen,), jnp.int32)])

    degp = _sc_deg(dst_p).reshape(2, NPAD, DD)[:, :NN]
    gidx = _gidx_build(src_p, et_p)

    z0 = _zbuild(x, basis_0, coeff_0)
    aggp0 = _sc_agg(z0, gidx, dst_p).reshape(2, NPAD, DD)[:, :NN]
    h0, ft = _dense0(aggp0, degp, x, prev_embed_0, time_diff,
                     wself_0, bias_0, gru_wx_0, gru_wh_0, gru_bx_0, gru_bh_0)

    z1 = _zbuild(ft, basis_1, coeff_1)
    aggp1 = _sc_agg(z1, gidx, dst_p).reshape(2, NPAD, DD)[:, :NN]
    h1, final = _dense1(aggp1, degp, ft, prev_embed_1, time_diff, h0,
                        lf_alpha, wself_1, bias_1, gru_wx_1, gru_wh_1,
                        gru_bx_1, gru_bh_1)

    return (jnp.stack([h0, h1], axis=0), final)
